# baseline (speedup 1.0000x reference)
import jax
import jax.numpy as jnp
from jax.experimental import pallas as pl
from jax.experimental.pallas import tpu as pltpu

NODE_NUM = 100
HIDDEN_SIZE = 4096
NCHUNK = 8
CHUNK_COLS = HIDDEN_SIZE // NCHUNK


def _dma_body(in_hbm, out_hbm, buf, insem, outsem):
    for c in range(NCHUNK):
        pltpu.make_async_copy(
            in_hbm.at[:, pl.ds(c * CHUNK_COLS, CHUNK_COLS)],
            buf.at[:, pl.ds(c * CHUNK_COLS, CHUNK_COLS)],
            insem.at[c],
        ).start()
    for c in range(NCHUNK):
        pltpu.make_async_copy(
            in_hbm.at[:, pl.ds(c * CHUNK_COLS, CHUNK_COLS)],
            buf.at[:, pl.ds(c * CHUNK_COLS, CHUNK_COLS)],
            insem.at[c],
        ).wait()
        pltpu.make_async_copy(
            buf.at[:, pl.ds(c * CHUNK_COLS, CHUNK_COLS)],
            out_hbm.at[:, pl.ds(c * CHUNK_COLS, CHUNK_COLS)],
            outsem.at[c],
        ).start()
    for c in range(NCHUNK):
        pltpu.make_async_copy(
            buf.at[:, pl.ds(c * CHUNK_COLS, CHUNK_COLS)],
            out_hbm.at[:, pl.ds(c * CHUNK_COLS, CHUNK_COLS)],
            outsem.at[c],
        ).wait()


def kernel(node_table):
    return pl.pallas_call(
        _dma_body,
        out_shape=jax.ShapeDtypeStruct((NODE_NUM, HIDDEN_SIZE), jnp.float32),
        in_specs=[pl.BlockSpec(memory_space=pl.ANY)],
        out_specs=pl.BlockSpec(memory_space=pl.ANY),
        scratch_shapes=[
            pltpu.VMEM((NODE_NUM, HIDDEN_SIZE), jnp.float32),
            pltpu.SemaphoreType.DMA((NCHUNK,)),
            pltpu.SemaphoreType.DMA((NCHUNK,)),
        ],
    )(node_table)
